# Initial kernel scaffold; baseline (speedup 1.0000x reference)
import jax
import jax.numpy as jnp
from jax.experimental import pallas as pl


def kernel(x, emb_res, emb_inc, emb_typ, emb_wrk, W, b):
    raise NotImplementedError("write your pallas kernel here")


# speedup vs baseline: 3.9582x; 3.9582x over previous
import functools

import jax
import jax.numpy as jnp
from jax import lax
from jax.experimental import pallas as pl
from jax.experimental.pallas import tpu as pltpu
from jax.experimental.pallas import tpu_sc as plsc

EMBED_DIM = 128
BATCH = 16384
NUM_CORES = 2
NUM_SUBCORES = 16
NUM_WORKERS = NUM_CORES * NUM_SUBCORES
ROWS_PER_W = BATCH // NUM_WORKERS
CHUNK = 128
NCHUNK = ROWS_PER_W // CHUNK
LANES = 16


def _build_tables_body(res_ref, inc_ref, typ_ref, wrk_ref, w_ref, b_ref,
                       t12_ref, t34_ref):
    w = w_ref[...]
    t1 = jnp.dot(res_ref[...], w[0:8, :], preferred_element_type=jnp.float32)
    t2 = jnp.dot(inc_ref[...], w[8:24, :], preferred_element_type=jnp.float32)
    t3 = jnp.dot(typ_ref[...], w[24:32, :], preferred_element_type=jnp.float32)
    t4 = jnp.dot(wrk_ref[...], w[32:40, :], preferred_element_type=jnp.float32)
    bias = b_ref[...]
    for i in range(20):
        t12_ref[pl.ds(i * 20, 20), :] = t1[i:i + 1, :] + t2
    t4b = t4 + bias
    for i in range(10):
        t34_ref[pl.ds(i * 10, 10), :] = t3[i:i + 1, :] + t4b


def _build_tables(emb_res, emb_inc, emb_typ, emb_wrk, W, b):
    return pl.pallas_call(
        _build_tables_body,
        out_shape=(
            jax.ShapeDtypeStruct((400, EMBED_DIM), jnp.float32),
            jax.ShapeDtypeStruct((100, EMBED_DIM), jnp.float32),
        ),
    )(emb_res, emb_inc, emb_typ, emb_wrk, W, b.reshape(1, EMBED_DIM))


def _sc_lookup(x0, x1, x2, x3, t12, t34):
    mesh = plsc.VectorSubcoreMesh(core_axis_name="c", subcore_axis_name="s")

    @functools.partial(
        pl.kernel,
        mesh=mesh,
        out_type=jax.ShapeDtypeStruct((BATCH, EMBED_DIM), jnp.float32),
        scratch_types=[
            pltpu.VMEM((ROWS_PER_W,), jnp.int32),
            pltpu.VMEM((ROWS_PER_W,), jnp.int32),
            pltpu.VMEM((ROWS_PER_W,), jnp.int32),
            pltpu.VMEM((ROWS_PER_W,), jnp.int32),
            pltpu.VMEM((NCHUNK, CHUNK), jnp.int32),
            pltpu.VMEM((NCHUNK, CHUNK), jnp.int32),
            pltpu.VMEM((CHUNK, EMBED_DIM), jnp.float32),
            pltpu.VMEM((CHUNK, EMBED_DIM), jnp.float32),
            pltpu.SemaphoreType.DMA,
            pltpu.SemaphoreType.DMA,
        ],
    )
    def k(x0h, x1h, x2h, x3h, t12h, t34h, outh,
          x0v, x1v, x2v, x3v, i12v, i34v, b12, b34, s1, s2):
        wid = lax.axis_index("s") * NUM_CORES + lax.axis_index("c")
        base = wid * ROWS_PER_W
        pltpu.sync_copy(x0h.at[pl.ds(base, ROWS_PER_W)], x0v)
        pltpu.sync_copy(x1h.at[pl.ds(base, ROWS_PER_W)], x1v)
        pltpu.sync_copy(x2h.at[pl.ds(base, ROWS_PER_W)], x2v)
        pltpu.sync_copy(x3h.at[pl.ds(base, ROWS_PER_W)], x3v)
        per_chunk = CHUNK // LANES
        for j in range(ROWS_PER_W // LANES):
            a0 = x0v[pl.ds(j * LANES, LANES)]
            a1 = x1v[pl.ds(j * LANES, LANES)]
            a2 = x2v[pl.ds(j * LANES, LANES)]
            a3 = x3v[pl.ds(j * LANES, LANES)]
            c = j // per_chunk
            o = (j % per_chunk) * LANES
            i12v[c, pl.ds(o, LANES)] = a0 * 20 + a1
            i34v[c, pl.ds(o, LANES)] = a2 * 10 + a3
        for c in range(NCHUNK):
            cp1 = pltpu.async_copy(t12h.at[i12v.at[c]], b12, s1)
            cp2 = pltpu.async_copy(t34h.at[i34v.at[c]], b34, s2)
            cp1.wait()
            cp2.wait()

            def add_body(r, carry):
                for u in range(4):
                    for jj in range(EMBED_DIM // LANES):
                        sl = pl.ds(jj * LANES, LANES)
                        b12[r * 4 + u, sl] = b12[r * 4 + u, sl] + b34[r * 4 + u, sl]
                return carry

            lax.fori_loop(0, CHUNK // 4, add_body, 0)
            pltpu.sync_copy(b12, outh.at[pl.ds(base + c * CHUNK, CHUNK)])

    return k(x0, x1, x2, x3, t12, t34)


def kernel(x, emb_res, emb_inc, emb_typ, emb_wrk, W, b):
    t12, t34 = _build_tables(emb_res, emb_inc, emb_typ, emb_wrk, W, b)
    xi = x.astype(jnp.int32)
    out = _sc_lookup(xi[:, 0], xi[:, 1], xi[:, 2], xi[:, 3], t12, t34)
    return out[:, None, :]


# speedup vs baseline: 4.0316x; 1.0185x over previous
import functools

import jax
import jax.numpy as jnp
from jax import lax
from jax.experimental import pallas as pl
from jax.experimental.pallas import tpu as pltpu
from jax.experimental.pallas import tpu_sc as plsc

EMBED_DIM = 128
BATCH = 16384
NUM_CORES = 2
NUM_SUBCORES = 16
NUM_WORKERS = NUM_CORES * NUM_SUBCORES
ROWS_PER_W = BATCH // NUM_WORKERS
CHUNK = 128
NCHUNK = ROWS_PER_W // CHUNK
LANES = 16


def _build_tables_body(res_ref, inc_ref, typ_ref, wrk_ref, w_ref, b_ref,
                       t12_ref, t34_ref):
    w = w_ref[...]
    t1 = jnp.dot(res_ref[...], w[0:8, :], preferred_element_type=jnp.float32)
    t2 = jnp.dot(inc_ref[...], w[8:24, :], preferred_element_type=jnp.float32)
    t3 = jnp.dot(typ_ref[...], w[24:32, :], preferred_element_type=jnp.float32)
    t4 = jnp.dot(wrk_ref[...], w[32:40, :], preferred_element_type=jnp.float32)
    bias = b_ref[...]
    for i in range(20):
        t12_ref[pl.ds(i * 20, 20), :] = t1[i:i + 1, :] + t2
    t4b = t4 + bias
    for i in range(10):
        t34_ref[pl.ds(i * 10, 10), :] = t3[i:i + 1, :] + t4b


def _build_tables(emb_res, emb_inc, emb_typ, emb_wrk, W, b):
    return pl.pallas_call(
        _build_tables_body,
        out_shape=(
            jax.ShapeDtypeStruct((400, EMBED_DIM), jnp.float32),
            jax.ShapeDtypeStruct((100, EMBED_DIM), jnp.float32),
        ),
    )(emb_res, emb_inc, emb_typ, emb_wrk, W, b.reshape(1, EMBED_DIM))


def _sc_lookup(x0, x1, x2, x3, t12, t34):
    mesh = plsc.VectorSubcoreMesh(core_axis_name="c", subcore_axis_name="s")

    @functools.partial(
        pl.kernel,
        mesh=mesh,
        out_type=jax.ShapeDtypeStruct((BATCH, EMBED_DIM), jnp.float32),
        scratch_types=[
            pltpu.VMEM((ROWS_PER_W,), jnp.int32),
            pltpu.VMEM((ROWS_PER_W,), jnp.int32),
            pltpu.VMEM((ROWS_PER_W,), jnp.int32),
            pltpu.VMEM((ROWS_PER_W,), jnp.int32),
            pltpu.VMEM((NCHUNK, CHUNK), jnp.int32),
            pltpu.VMEM((NCHUNK, CHUNK), jnp.int32),
            pltpu.VMEM((CHUNK, EMBED_DIM), jnp.float32),
            pltpu.VMEM((CHUNK, EMBED_DIM), jnp.float32),
            pltpu.VMEM((CHUNK, EMBED_DIM), jnp.float32),
            pltpu.VMEM((CHUNK, EMBED_DIM), jnp.float32),
            pltpu.SemaphoreType.DMA,
            pltpu.SemaphoreType.DMA,
            pltpu.SemaphoreType.DMA,
            pltpu.SemaphoreType.DMA,
        ],
    )
    def k(x0h, x1h, x2h, x3h, t12h, t34h, outh,
          x0v, x1v, x2v, x3v, i12v, i34v,
          b12a, b12b, b34a, b34b, sga, sgb, soa, sob):
        b12s, b34s = [b12a, b12b], [b34a, b34b]
        sg, so = [sga, sgb], [soa, sob]
        wid = lax.axis_index("s") * NUM_CORES + lax.axis_index("c")
        base = wid * ROWS_PER_W
        pltpu.sync_copy(x0h.at[pl.ds(base, ROWS_PER_W)], x0v)
        pltpu.sync_copy(x1h.at[pl.ds(base, ROWS_PER_W)], x1v)
        pltpu.sync_copy(x2h.at[pl.ds(base, ROWS_PER_W)], x2v)
        pltpu.sync_copy(x3h.at[pl.ds(base, ROWS_PER_W)], x3v)
        per_chunk = CHUNK // LANES
        for j in range(ROWS_PER_W // LANES):
            a0 = x0v[pl.ds(j * LANES, LANES)]
            a1 = x1v[pl.ds(j * LANES, LANES)]
            a2 = x2v[pl.ds(j * LANES, LANES)]
            a3 = x3v[pl.ds(j * LANES, LANES)]
            c = j // per_chunk
            o = (j % per_chunk) * LANES
            i12v[c, pl.ds(o, LANES)] = a0 * 20 + a1
            i34v[c, pl.ds(o, LANES)] = a2 * 10 + a3

        def issue(c):
            s = c % 2
            return (pltpu.async_copy(t12h.at[i12v.at[c]], b12s[s], sg[s]),
                    pltpu.async_copy(t34h.at[i34v.at[c]], b34s[s], sg[s]))

        UNROLL = 4
        gcp = [None] * NCHUNK
        ocp = [None] * NCHUNK
        gcp[0] = issue(0)
        for c in range(NCHUNK):
            s = c % 2
            if c + 1 < NCHUNK:
                if c >= 1:
                    ocp[c - 1].wait()
                gcp[c + 1] = issue(c + 1)
            gcp[c][0].wait()
            gcp[c][1].wait()
            b12, b34 = b12s[s], b34s[s]

            def add_body(r, carry):
                for u in range(UNROLL):
                    for jj in range(EMBED_DIM // LANES):
                        sl = pl.ds(jj * LANES, LANES)
                        plsc.addupdate(b12.at[r * UNROLL + u, sl],
                                       b34[r * UNROLL + u, sl])
                return carry

            lax.fori_loop(0, CHUNK // UNROLL, add_body, 0)
            ocp[c] = pltpu.async_copy(
                b12, outh.at[pl.ds(base + c * CHUNK, CHUNK)], so[s])
        ocp[NCHUNK - 2].wait()
        ocp[NCHUNK - 1].wait()

    return k(x0, x1, x2, x3, t12, t34)


def kernel(x, emb_res, emb_inc, emb_typ, emb_wrk, W, b):
    t12, t34 = _build_tables(emb_res, emb_inc, emb_typ, emb_wrk, W, b)
    xi = x.astype(jnp.int32)
    out = _sc_lookup(xi[:, 0], xi[:, 1], xi[:, 2], xi[:, 3], t12, t34)
    return out[:, None, :]


# speedup vs baseline: 5.7917x; 1.4366x over previous
import functools

import jax
import jax.numpy as jnp
from jax import lax
from jax.experimental import pallas as pl
from jax.experimental.pallas import tpu as pltpu
from jax.experimental.pallas import tpu_sc as plsc

EMBED_DIM = 128
BATCH = 16384
NUM_CORES = 2
NUM_SUBCORES = 16
NUM_WORKERS = NUM_CORES * NUM_SUBCORES
ROWS_PER_W = BATCH // NUM_WORKERS
CHUNK = 128
NCHUNK = ROWS_PER_W // CHUNK
LANES = 16


def _build_tables_body(res_ref, inc_ref, typ_ref, wrk_ref, w_ref, b_ref,
                       t12_ref, t34_ref):
    w = w_ref[...]
    t1 = jnp.dot(res_ref[...], w[0:8, :], preferred_element_type=jnp.float32)
    t2 = jnp.dot(inc_ref[...], w[8:24, :], preferred_element_type=jnp.float32)
    t3 = jnp.dot(typ_ref[...], w[24:32, :], preferred_element_type=jnp.float32)
    t4 = jnp.dot(wrk_ref[...], w[32:40, :], preferred_element_type=jnp.float32)
    bias = b_ref[...]
    for i in range(20):
        t12_ref[pl.ds(i * 20, 20), :] = t1[i:i + 1, :] + t2
    t4b = t4 + bias
    for i in range(10):
        t34_ref[pl.ds(i * 10, 10), :] = t3[i:i + 1, :] + t4b


def _build_tables(emb_res, emb_inc, emb_typ, emb_wrk, W, b):
    return pl.pallas_call(
        _build_tables_body,
        out_shape=(
            jax.ShapeDtypeStruct((400, EMBED_DIM), jnp.float32),
            jax.ShapeDtypeStruct((100, EMBED_DIM), jnp.float32),
        ),
    )(emb_res, emb_inc, emb_typ, emb_wrk, W, b.reshape(1, EMBED_DIM))


def _sc_lookup(x0, x1, x2, x3, t12, t34):
    mesh = plsc.VectorSubcoreMesh(core_axis_name="c", subcore_axis_name="s")

    @functools.partial(
        pl.kernel,
        mesh=mesh,
        out_type=jax.ShapeDtypeStruct((BATCH, EMBED_DIM), jnp.float32),
        scratch_types=[
            pltpu.VMEM((ROWS_PER_W,), jnp.int32),
            pltpu.VMEM((ROWS_PER_W,), jnp.int32),
            pltpu.VMEM((ROWS_PER_W,), jnp.int32),
            pltpu.VMEM((ROWS_PER_W,), jnp.int32),
            pltpu.VMEM((NCHUNK, CHUNK), jnp.int32),
            pltpu.VMEM((NCHUNK, CHUNK), jnp.int32),
            pltpu.VMEM((CHUNK, EMBED_DIM), jnp.float32),
            pltpu.VMEM((CHUNK, EMBED_DIM), jnp.float32),
            pltpu.VMEM((CHUNK, EMBED_DIM), jnp.float32),
            pltpu.VMEM((CHUNK, EMBED_DIM), jnp.float32),
            pltpu.VMEM_SHARED((400, EMBED_DIM), jnp.float32),
            pltpu.VMEM_SHARED((100, EMBED_DIM), jnp.float32),
            pltpu.SemaphoreType.DMA,
            pltpu.SemaphoreType.DMA,
            pltpu.SemaphoreType.DMA,
            pltpu.SemaphoreType.DMA,
        ],
    )
    def k(x0h, x1h, x2h, x3h, t12h, t34h, outh,
          x0v, x1v, x2v, x3v, i12v, i34v,
          b12a, b12b, b34a, b34b, t12s, t34s, sga, sgb, soa, sob):
        b12s, b34s = [b12a, b12b], [b34a, b34b]
        sg, so = [sga, sgb], [soa, sob]
        sid = lax.axis_index("s")
        wid = sid * NUM_CORES + lax.axis_index("c")
        base = wid * ROWS_PER_W

        @pl.when(sid == 0)
        def _stage_tables():
            pltpu.sync_copy(t12h, t12s)
            pltpu.sync_copy(t34h, t34s)
        pltpu.sync_copy(x0h.at[pl.ds(base, ROWS_PER_W)], x0v)
        pltpu.sync_copy(x1h.at[pl.ds(base, ROWS_PER_W)], x1v)
        pltpu.sync_copy(x2h.at[pl.ds(base, ROWS_PER_W)], x2v)
        pltpu.sync_copy(x3h.at[pl.ds(base, ROWS_PER_W)], x3v)
        per_chunk = CHUNK // LANES
        for j in range(ROWS_PER_W // LANES):
            a0 = x0v[pl.ds(j * LANES, LANES)]
            a1 = x1v[pl.ds(j * LANES, LANES)]
            a2 = x2v[pl.ds(j * LANES, LANES)]
            a3 = x3v[pl.ds(j * LANES, LANES)]
            c = j // per_chunk
            o = (j % per_chunk) * LANES
            i12v[c, pl.ds(o, LANES)] = a0 * 20 + a1
            i34v[c, pl.ds(o, LANES)] = a2 * 10 + a3

        plsc.subcore_barrier()

        def issue(c):
            s = c % 2
            return (pltpu.async_copy(t12s.at[i12v.at[c]], b12s[s], sg[s]),
                    pltpu.async_copy(t34s.at[i34v.at[c]], b34s[s], sg[s]))

        UNROLL = 4
        gcp = [None] * NCHUNK
        ocp = [None] * NCHUNK
        gcp[0] = issue(0)
        for c in range(NCHUNK):
            s = c % 2
            if c + 1 < NCHUNK:
                if c >= 1:
                    ocp[c - 1].wait()
                gcp[c + 1] = issue(c + 1)
            gcp[c][0].wait()
            gcp[c][1].wait()
            b12, b34 = b12s[s], b34s[s]

            def add_body(r, carry):
                for u in range(UNROLL):
                    for jj in range(EMBED_DIM // LANES):
                        sl = pl.ds(jj * LANES, LANES)
                        plsc.addupdate(b12.at[r * UNROLL + u, sl],
                                       b34[r * UNROLL + u, sl])
                return carry

            lax.fori_loop(0, CHUNK // UNROLL, add_body, 0)
            ocp[c] = pltpu.async_copy(
                b12, outh.at[pl.ds(base + c * CHUNK, CHUNK)], so[s])
        ocp[NCHUNK - 2].wait()
        ocp[NCHUNK - 1].wait()

    return k(x0, x1, x2, x3, t12, t34)


def kernel(x, emb_res, emb_inc, emb_typ, emb_wrk, W, b):
    t12, t34 = _build_tables(emb_res, emb_inc, emb_typ, emb_wrk, W, b)
    xi = x.astype(jnp.int32)
    out = _sc_lookup(xi[:, 0], xi[:, 1], xi[:, 2], xi[:, 3], t12, t34)
    return out[:, None, :]


# speedup vs baseline: 6.3293x; 1.0928x over previous
import functools

import jax
import jax.numpy as jnp
from jax import lax
from jax.experimental import pallas as pl
from jax.experimental.pallas import tpu as pltpu
from jax.experimental.pallas import tpu_sc as plsc

EMBED_DIM = 128
BATCH = 16384
NUM_CORES = 2
NUM_SUBCORES = 16
NUM_WORKERS = NUM_CORES * NUM_SUBCORES
ROWS_PER_W = BATCH // NUM_WORKERS
CHUNK = 128
NCHUNK = ROWS_PER_W // CHUNK
LANES = 16


def _build_tables_body(xt_ref, res_ref, inc_ref, typ_ref, wrk_ref, w_ref, b_ref,
                       t12_ref, t34_ref, c12_ref, c34_ref):
    xt = xt_ref[...]
    c12_ref[...] = xt[0:1, :] * 20 + xt[1:2, :]
    c34_ref[...] = xt[2:3, :] * 10 + xt[3:4, :]
    w = w_ref[...]
    t1 = jnp.dot(res_ref[...], w[0:8, :], preferred_element_type=jnp.float32)
    t2 = jnp.dot(inc_ref[...], w[8:24, :], preferred_element_type=jnp.float32)
    t3 = jnp.dot(typ_ref[...], w[24:32, :], preferred_element_type=jnp.float32)
    t4 = jnp.dot(wrk_ref[...], w[32:40, :], preferred_element_type=jnp.float32)
    bias = b_ref[...]
    for i in range(20):
        t12_ref[pl.ds(i * 20, 20), :] = t1[i:i + 1, :] + t2
    t4b = t4 + bias
    for i in range(10):
        t34_ref[pl.ds(i * 10, 10), :] = t3[i:i + 1, :] + t4b


def _build_tables(x, emb_res, emb_inc, emb_typ, emb_wrk, W, b):
    return pl.pallas_call(
        _build_tables_body,
        out_shape=(
            jax.ShapeDtypeStruct((400, EMBED_DIM), jnp.float32),
            jax.ShapeDtypeStruct((100, EMBED_DIM), jnp.float32),
            jax.ShapeDtypeStruct((1, BATCH), jnp.int32),
            jax.ShapeDtypeStruct((1, BATCH), jnp.int32),
        ),
    )(x, emb_res, emb_inc, emb_typ, emb_wrk, W, b.reshape(1, EMBED_DIM))


def _sc_lookup(c12, c34, t12, t34):
    mesh = plsc.VectorSubcoreMesh(core_axis_name="c", subcore_axis_name="s")

    @functools.partial(
        pl.kernel,
        mesh=mesh,
        out_type=jax.ShapeDtypeStruct((BATCH, EMBED_DIM), jnp.float32),
        scratch_types=[
            pltpu.VMEM((ROWS_PER_W,), jnp.int32),
            pltpu.VMEM((ROWS_PER_W,), jnp.int32),
            pltpu.VMEM((CHUNK, EMBED_DIM), jnp.float32),
            pltpu.VMEM((CHUNK, EMBED_DIM), jnp.float32),
            pltpu.VMEM((CHUNK, EMBED_DIM), jnp.float32),
            pltpu.VMEM((CHUNK, EMBED_DIM), jnp.float32),
            pltpu.VMEM_SHARED((400, EMBED_DIM), jnp.float32),
            pltpu.VMEM_SHARED((100, EMBED_DIM), jnp.float32),
            pltpu.SemaphoreType.DMA,
            pltpu.SemaphoreType.DMA,
            pltpu.SemaphoreType.DMA,
            pltpu.SemaphoreType.DMA,
        ],
    )
    def k(c12h, c34h, t12h, t34h, outh,
          i12v, i34v,
          b12a, b12b, b34a, b34b, t12s, t34s, sga, sgb, soa, sob):
        b12s, b34s = [b12a, b12b], [b34a, b34b]
        sg, so = [sga, sgb], [soa, sob]
        sid = lax.axis_index("s")
        wid = sid * NUM_CORES + lax.axis_index("c")
        base = wid * ROWS_PER_W

        @pl.when(sid == 0)
        def _stage_tables():
            pltpu.sync_copy(t12h, t12s)
            pltpu.sync_copy(t34h, t34s)
        pltpu.sync_copy(c12h.at[pl.ds(base, ROWS_PER_W)], i12v)
        pltpu.sync_copy(c34h.at[pl.ds(base, ROWS_PER_W)], i34v)

        plsc.subcore_barrier()

        def issue(c):
            s = c % 2
            isl = pl.ds(c * CHUNK, CHUNK)
            return (pltpu.async_copy(t12s.at[i12v.at[isl]], b12s[s], sg[s]),
                    pltpu.async_copy(t34s.at[i34v.at[isl]], b34s[s], sg[s]))

        UNROLL = 4
        gcp = [None] * NCHUNK
        ocp = [None] * NCHUNK
        gcp[0] = issue(0)
        for c in range(NCHUNK):
            s = c % 2
            if c + 1 < NCHUNK:
                if c >= 1:
                    ocp[c - 1].wait()
                gcp[c + 1] = issue(c + 1)
            gcp[c][0].wait()
            gcp[c][1].wait()
            b12, b34 = b12s[s], b34s[s]

            def add_body(r, carry):
                for u in range(UNROLL):
                    for jj in range(EMBED_DIM // LANES):
                        sl = pl.ds(jj * LANES, LANES)
                        plsc.addupdate(b12.at[r * UNROLL + u, sl],
                                       b34[r * UNROLL + u, sl])
                return carry

            lax.fori_loop(0, CHUNK // UNROLL, add_body, 0)
            ocp[c] = pltpu.async_copy(
                b12, outh.at[pl.ds(base + c * CHUNK, CHUNK)], so[s])
        ocp[NCHUNK - 2].wait()
        ocp[NCHUNK - 1].wait()

    return k(c12, c34, t12, t34)


def kernel(x, emb_res, emb_inc, emb_typ, emb_wrk, W, b):
    t12, t34, c12, c34 = _build_tables(
        x.astype(jnp.int32).T, emb_res, emb_inc, emb_typ, emb_wrk, W, b)
    out = _sc_lookup(c12.reshape(BATCH), c34.reshape(BATCH), t12, t34)
    return out[:, None, :]


# speedup vs baseline: 6.4792x; 1.0237x over previous
import functools

import jax
import jax.numpy as jnp
from jax import lax
from jax.experimental import pallas as pl
from jax.experimental.pallas import tpu as pltpu
from jax.experimental.pallas import tpu_sc as plsc

EMBED_DIM = 128
BATCH = 16384
NUM_CORES = 2
NUM_SUBCORES = 16
NUM_WORKERS = NUM_CORES * NUM_SUBCORES
ROWS_PER_W = BATCH // NUM_WORKERS
CHUNK = 128
NCHUNK = ROWS_PER_W // CHUNK
LANES = 16


def _build_tables_body(res_ref, inc_ref, typ_ref, wrk_ref, w_ref, b_ref,
                       t12_ref, t34_ref):
    w = w_ref[...]
    t1 = jnp.dot(res_ref[...], w[0:8, :], preferred_element_type=jnp.float32)
    t2 = jnp.dot(inc_ref[...], w[8:24, :], preferred_element_type=jnp.float32)
    t3 = jnp.dot(typ_ref[...], w[24:32, :], preferred_element_type=jnp.float32)
    t4 = jnp.dot(wrk_ref[...], w[32:40, :], preferred_element_type=jnp.float32)
    bias = b_ref[...]
    for i in range(20):
        t12_ref[pl.ds(i * 20, 20), :] = t1[i:i + 1, :] + t2
    t4b = t4 + bias
    for i in range(10):
        t34_ref[pl.ds(i * 10, 10), :] = t3[i:i + 1, :] + t4b


def _build_tables(emb_res, emb_inc, emb_typ, emb_wrk, W, b):
    return pl.pallas_call(
        _build_tables_body,
        out_shape=(
            jax.ShapeDtypeStruct((400, EMBED_DIM), jnp.float32),
            jax.ShapeDtypeStruct((100, EMBED_DIM), jnp.float32),
        ),
    )(emb_res, emb_inc, emb_typ, emb_wrk, W, b.reshape(1, EMBED_DIM))


def _sc_lookup(xt, t12, t34):
    mesh = plsc.VectorSubcoreMesh(core_axis_name="c", subcore_axis_name="s")

    @functools.partial(
        pl.kernel,
        mesh=mesh,
        out_type=jax.ShapeDtypeStruct((BATCH, EMBED_DIM), jnp.float32),
        scratch_types=[
            pltpu.VMEM((ROWS_PER_W,), jnp.int32),
            pltpu.VMEM((ROWS_PER_W,), jnp.int32),
            pltpu.VMEM((ROWS_PER_W,), jnp.int32),
            pltpu.VMEM((ROWS_PER_W,), jnp.int32),
            pltpu.VMEM((ROWS_PER_W,), jnp.int32),
            pltpu.VMEM((ROWS_PER_W,), jnp.int32),
            pltpu.VMEM((CHUNK, EMBED_DIM), jnp.float32),
            pltpu.VMEM((CHUNK, EMBED_DIM), jnp.float32),
            pltpu.VMEM((CHUNK, EMBED_DIM), jnp.float32),
            pltpu.VMEM((CHUNK, EMBED_DIM), jnp.float32),
            pltpu.VMEM_SHARED((400, EMBED_DIM), jnp.float32),
            pltpu.VMEM_SHARED((100, EMBED_DIM), jnp.float32),
            pltpu.SemaphoreType.DMA,
            pltpu.SemaphoreType.DMA,
            pltpu.SemaphoreType.DMA,
            pltpu.SemaphoreType.DMA,
            pltpu.SemaphoreType.DMA,
        ],
    )
    def k(xth, t12h, t34h, outh,
          x0v, x1v, x2v, x3v, i12v, i34v,
          b12a, b12b, b34a, b34b, t12s, t34s, sx, sga, sgb, soa, sob):
        b12s, b34s = [b12a, b12b], [b34a, b34b]
        sg, so = [sga, sgb], [soa, sob]
        sid = lax.axis_index("s")
        wid = sid * NUM_CORES + lax.axis_index("c")
        base = wid * ROWS_PER_W

        xcp = [pltpu.async_copy(xth.at[f, pl.ds(base, ROWS_PER_W)], xv, sx)
               for f, xv in enumerate([x0v, x1v, x2v, x3v])]

        @pl.when(sid == 0)
        def _stage_tables():
            pltpu.sync_copy(t12h, t12s)
            pltpu.sync_copy(t34h, t34s)

        for c in xcp:
            c.wait()
        for r in range(ROWS_PER_W // LANES):
            sl = pl.ds(r * LANES, LANES)
            i12v[sl] = x0v[sl] * 20 + x1v[sl]
            i34v[sl] = x2v[sl] * 10 + x3v[sl]

        plsc.subcore_barrier()

        def issue(c):
            s = c % 2
            isl = pl.ds(c * CHUNK, CHUNK)
            return (pltpu.async_copy(t12s.at[i12v.at[isl]], b12s[s], sg[s]),
                    pltpu.async_copy(t34s.at[i34v.at[isl]], b34s[s], sg[s]))

        UNROLL = 4
        gcp = [None] * NCHUNK
        ocp = [None] * NCHUNK
        gcp[0] = issue(0)
        for c in range(NCHUNK):
            s = c % 2
            if c + 1 < NCHUNK:
                if c >= 1:
                    ocp[c - 1].wait()
                gcp[c + 1] = issue(c + 1)
            gcp[c][0].wait()
            gcp[c][1].wait()
            b12, b34 = b12s[s], b34s[s]

            def add_body(r, carry):
                for u in range(UNROLL):
                    for jj in range(EMBED_DIM // LANES):
                        sl = pl.ds(jj * LANES, LANES)
                        plsc.addupdate(b12.at[r * UNROLL + u, sl],
                                       b34[r * UNROLL + u, sl])
                return carry

            lax.fori_loop(0, CHUNK // UNROLL, add_body, 0)
            ocp[c] = pltpu.async_copy(
                b12, outh.at[pl.ds(base + c * CHUNK, CHUNK)], so[s])
        ocp[NCHUNK - 2].wait()
        ocp[NCHUNK - 1].wait()

    return k(xt, t12, t34)


def kernel(x, emb_res, emb_inc, emb_typ, emb_wrk, W, b):
    t12, t34 = _build_tables(emb_res, emb_inc, emb_typ, emb_wrk, W, b)
    out = _sc_lookup(x.astype(jnp.int32).T, t12, t34)
    return out[:, None, :]
